# baseline (speedup 1.0000x reference)
import functools

import jax
import jax.numpy as jnp
from jax import lax
from jax.experimental import pallas as pl
from jax.experimental.pallas import tpu as pltpu
from jax.experimental.pallas import tpu_sc as plsc

N = 100000
D = 256
B = 16384
RAW = 4 * D
HID = 2 * D
MSG = 100

GRID = 32
BB = B // GRID
MB = 3200

NC = 2
NS = 16
NW = NC * NS
BPW = B // NW
GC = 128
NCH = BPW // GC
SPC = (B // 16) // NCH
RPW = N // NW
SCCH = 192
WL = 18432
EBITS = 14
INVALID = 0x7FFFFFFF

_sc_mesh = plsc.VectorSubcoreMesh(core_axis_name="c", subcore_axis_name="s")


@functools.partial(
    pl.kernel,
    out_type=(
        jax.ShapeDtypeStruct((B, D), jnp.float32),
        jax.ShapeDtypeStruct((NW * WL,), jnp.int32),
    ),
    mesh=_sc_mesh,
    scratch_types=[
        pltpu.VMEM((B,), jnp.int32),
        pltpu.VMEM((B + 16,), jnp.int32),
        pltpu.VMEM((WL,), jnp.int32),
        pltpu.VMEM((RPW + 16,), jnp.int32),
        pltpu.VMEM((GC, D), jnp.float32),
        pltpu.VMEM((GC, D), jnp.float32),
        pltpu.SemaphoreType.DMA,
        pltpu.SemaphoreType.DMA,
    ],
    compiler_params=pltpu.CompilerParams(needs_layout_passes=False),
)
def _sc_prep(mem_hbm, idx_hbm, h_hbm, wl_hbm, idx_v, plist, wlist, claim,
             buf0, buf1, gsem, osem):
    wid = lax.axis_index("s") * NC + lax.axis_index("c")
    base = wid * BPW
    lo = wid * RPW
    hi = lo + RPW
    lane = lax.iota(jnp.int32, 16)

    pltpu.sync_copy(idx_hbm, idx_v)

    def scan_body(j, cnt):
        v = idx_v[pl.ds(j * 16, 16)]
        m = (v >= lo) & (v < hi)
        pack = ((v - lo) << EBITS) | (lane + j * 16)
        c = plsc.cumsum(jnp.where(m, jnp.int32(1), jnp.int32(0)))
        plsc.store_scatter(plist, [cnt + c - 1], pack, mask=m)
        return cnt + c[15]

    bufs = (buf0, buf1)
    cnt = jnp.int32(0)
    outs = [None] * NCH
    for c in range(NCH):
        if c >= 2:
            outs[c - 2].wait()
        g = pltpu.make_async_copy(
            mem_hbm.at[idx_v.at[pl.ds(base + c * GC, GC)]], bufs[c % 2], gsem)
        g.start()
        cnt = lax.fori_loop(c * SPC, (c + 1) * SPC, scan_body, cnt)
        g.wait()
        o = pltpu.make_async_copy(
            bufs[c % 2], h_hbm.at[pl.ds(base + c * GC, GC)], osem)
        o.start()
        outs[c] = o

    nchunk = (cnt + 15) // 16

    def claim_body(t, _):
        p = t * 16
        pk = plist[pl.ds(p, 16)]
        valid = lane < (cnt - p)
        pk = jnp.where(valid, pk, INVALID)
        sk, _sv = plsc.sort_key_val(pk, pk)
        grp = lax.shift_right_logical(sk, EBITS)
        nxt = grp[jnp.minimum(lane + 1, 15)]
        win = ((grp != nxt) | (lane == 15)) & (sk != INVALID)
        plsc.store_scatter(claim, [grp], sk & (B - 1), mask=win)
        return 0

    lax.fori_loop(0, nchunk, claim_body, 0)

    def winner_body(t, wcnt):
        p = t * 16
        pk = plist[pl.ds(p, 16)]
        valid = lane < (cnt - p)
        rrel = jnp.where(valid, lax.shift_right_logical(pk, EBITS), 0)
        cl = plsc.load_gather(claim, [rrel])
        win = valid & (cl == (pk & (B - 1)))
        wc = plsc.cumsum(jnp.where(win, jnp.int32(1), jnp.int32(0)))
        plsc.store_scatter(wlist, [16 + wcnt + wc - 1], pk, mask=win)
        return wcnt + wc[15]

    wcnt = lax.fori_loop(0, nchunk, winner_body, jnp.int32(0))
    wlist[pl.ds(0, 16)] = jnp.broadcast_to(wcnt, (16,))

    pad = jnp.broadcast_to(wlist[pl.ds(16, 16)][0], (16,))

    def pad_body(t, _):
        wlist[pl.ds(16 + wcnt + t * 16, 16)] = pad
        return 0

    lax.fori_loop(0, 2 * SCCH // 16, pad_body, 0)

    nwords = 16 + ((wcnt + 2 * SCCH - 1) // (2 * SCCH)) * (2 * SCCH)

    def dump_body(t, _):
        pltpu.sync_copy(wlist.at[pl.ds(t * 2048, 2048)],
                        wl_hbm.at[pl.ds(wid * WL + t * 2048, 2048)])
        return 0

    lax.fori_loop(0, (nwords + 2047) // 2048, dump_body, 0)

    for c in range(max(0, NCH - 2), NCH):
        outs[c].wait()


def _tc_body(raw_ref, h_ref, mem_ref, W1_ref, b1_ref, W2_ref, b2_ref,
             Wx_ref, Wh_ref, bx_ref, bh_ref, out_mem_ref, h_new_ref):
    out_mem_ref[...] = mem_ref[...]
    f32 = jnp.float32
    bf = jnp.bfloat16
    x = jnp.maximum(
        lax.dot(raw_ref[...].astype(bf), W1_ref[...].astype(bf),
                preferred_element_type=f32) + b1_ref[...], 0.0)
    msg = lax.dot(x.astype(bf), W2_ref[...].astype(bf),
                  preferred_element_type=f32) + b2_ref[...]
    gx = lax.dot(msg.astype(bf), Wx_ref[...].astype(bf),
                 preferred_element_type=f32) + bx_ref[...]
    h = h_ref[...]
    gh = lax.dot(h.astype(bf), Wh_ref[...].astype(bf),
                 preferred_element_type=f32) + bh_ref[...]
    xr, xz, xn = gx[:, :D], gx[:, D:2 * D], gx[:, 2 * D:]
    hr, hz, hn = gh[:, :D], gh[:, D:2 * D], gh[:, 2 * D:]
    r = jax.nn.sigmoid(xr + hr)
    z = jax.nn.sigmoid(xz + hz)
    n = jnp.tanh(xn + r * hn)
    h_new_ref[...] = (1.0 - z) * n + z * h


def _tc_call(raw_msg, h, mem, W1, b1, W2, b2, Wx, Wh, bx, bh):
    full = lambda s: pl.BlockSpec(s, lambda b: (0, 0))
    return pl.pallas_call(
        _tc_body,
        grid=(GRID,),
        in_specs=[
            pl.BlockSpec((BB, RAW), lambda b: (b, 0)),
            pl.BlockSpec((BB, D), lambda b: (b, 0)),
            pl.BlockSpec((MB, D), lambda b: (b, 0)),
            full((RAW, HID)),
            full((1, HID)),
            full((HID, MSG)),
            full((1, MSG)),
            full((MSG, 3 * D)),
            full((D, 3 * D)),
            full((1, 3 * D)),
            full((1, 3 * D)),
        ],
        out_specs=[
            pl.BlockSpec((MB, D), lambda b: (b, 0)),
            pl.BlockSpec((BB, D), lambda b: (b, 0)),
        ],
        out_shape=[
            jax.ShapeDtypeStruct((N, D), jnp.float32),
            jax.ShapeDtypeStruct((B, D), jnp.float32),
        ],
        compiler_params=pltpu.CompilerParams(
            dimension_semantics=("arbitrary",),
        ),
    )(raw_msg, h, mem, W1, b1, W2, b2, Wx, Wh, bx, bh)


@functools.partial(
    pl.kernel,
    out_type=(),
    mesh=_sc_mesh,
    scratch_types=[
        pltpu.VMEM((16,), jnp.int32),
        pltpu.VMEM((SCCH,), jnp.int32),
        pltpu.VMEM((SCCH,), jnp.int32),
        pltpu.VMEM((SCCH,), jnp.int32),
        pltpu.VMEM((SCCH,), jnp.int32),
        pltpu.VMEM((SCCH,), jnp.int32),
        pltpu.VMEM((SCCH,), jnp.int32),
        pltpu.VMEM((SCCH, D), jnp.float32),
        pltpu.VMEM((SCCH, D), jnp.float32),
        pltpu.SemaphoreType.DMA,
        pltpu.SemaphoreType.DMA,
        pltpu.SemaphoreType.DMA,
        pltpu.SemaphoreType.DMA,
    ],
    compiler_params=pltpu.CompilerParams(needs_layout_passes=False),
)
def _sc_scatter(out_hbm, hnew_hbm, wl_hbm, hdr, wc0, wc1, ei0, ei1, ri0, ri1,
                rb0, rb1, gs0, gs1, ss0, ss1):
    wid = lax.axis_index("s") * NC + lax.axis_index("c")
    lo = wid * RPW

    pltpu.sync_copy(wl_hbm.at[pl.ds(wid * WL, 16)], hdr)
    wcnt = hdr[pl.ds(0, 16)][0]

    def pair_body(t, _):
        gathers = []
        for s, (wc, ei, ri, rb, gsem) in enumerate(
                ((wc0, ei0, ri0, rb0, gs0), (wc1, ei1, ri1, rb1, gs1))):
            off = wid * WL + 16 + (2 * t + s) * SCCH
            pltpu.sync_copy(wl_hbm.at[pl.ds(off, SCCH)], wc)

            def dec_body(k, _, wc=wc, ei=ei, ri=ri):
                pk = wc[pl.ds(k * 16, 16)]
                ei[pl.ds(k * 16, 16)] = pk & (B - 1)
                ri[pl.ds(k * 16, 16)] = lax.shift_right_logical(pk, EBITS) + lo
                return 0

            lax.fori_loop(0, SCCH // 16, dec_body, 0)
            g = pltpu.make_async_copy(hnew_hbm.at[ei], rb, gsem)
            g.start()
            gathers.append(g)
        scatters = []
        for (g, ri, rb, ssem) in ((gathers[0], ri0, rb0, ss0),
                                  (gathers[1], ri1, rb1, ss1)):
            g.wait()
            sc = pltpu.make_async_copy(rb, out_hbm.at[ri], ssem)
            sc.start()
            scatters.append(sc)
        for sc in scatters:
            sc.wait()
        return 0

    lax.fori_loop(0, (wcnt + 2 * SCCH - 1) // (2 * SCCH), pair_body, 0)


def kernel(mem, idx, raw_msg, W1, b1, W2, b2, Wx, Wh, bx, bh):
    h, wl = _sc_prep(mem, idx)

    out_mem, h_new = _tc_call(raw_msg, h, mem, W1, b1.reshape(1, -1), W2,
                              b2.reshape(1, -1), Wx, Wh, bx.reshape(1, -1),
                              bh.reshape(1, -1))

    out_ref = jax.new_ref(out_mem)
    _sc_scatter(out_ref, h_new, wl)
    return out_ref[...]


# speedup vs baseline: 1.2445x; 1.2445x over previous
import functools

import jax
import jax.numpy as jnp
from jax import lax
from jax.experimental import pallas as pl
from jax.experimental.pallas import tpu as pltpu
from jax.experimental.pallas import tpu_sc as plsc

N = 100000
D = 256
B = 16384
RAW = 4 * D
HID = 2 * D
MSG = 100

GRID = 32
BB = B // GRID
MB = 3200

NC = 2
NS = 16
NW = NC * NS
BPW = B // NW
GC = 64
NCH = BPW // GC
SPC = (B // 16) // NCH
RPW = N // NW
SCCH = 256
WL = 18432
EBITS = 14
INVALID = 0x7FFFFFFF

_sc_mesh = plsc.VectorSubcoreMesh(core_axis_name="c", subcore_axis_name="s")


@functools.partial(
    pl.kernel,
    out_type=(
        jax.ShapeDtypeStruct((B, D), jnp.float32),
        jax.ShapeDtypeStruct((NW * WL,), jnp.int32),
    ),
    mesh=_sc_mesh,
    scratch_types=[
        pltpu.VMEM((B,), jnp.int32),
        pltpu.VMEM((B + 16,), jnp.int32),
        pltpu.VMEM((WL,), jnp.int32),
        pltpu.VMEM((RPW + 16,), jnp.int32),
        pltpu.VMEM((GC, D), jnp.float32),
        pltpu.VMEM((GC, D), jnp.float32),
        pltpu.SemaphoreType.DMA,
        pltpu.SemaphoreType.DMA,
    ],
    compiler_params=pltpu.CompilerParams(needs_layout_passes=False),
)
def _sc_prep(mem_hbm, idx_hbm, h_hbm, wl_hbm, idx_v, plist, wlist, claim,
             buf0, buf1, gsem, osem):
    wid = lax.axis_index("s") * NC + lax.axis_index("c")
    base = wid * BPW
    lo = wid * RPW
    hi = lo + RPW
    lane = lax.iota(jnp.int32, 16)

    pltpu.sync_copy(idx_hbm, idx_v)

    def scan_body(j, cnt):
        v = idx_v[pl.ds(j * 16, 16)]
        m = (v >= lo) & (v < hi)
        pack = ((v - lo) << EBITS) | (lane + j * 16)
        c = plsc.cumsum(jnp.where(m, jnp.int32(1), jnp.int32(0)))
        plsc.store_scatter(plist, [cnt + c - 1], pack, mask=m)
        return cnt + c[15]

    bufs = (buf0, buf1)
    cnt = jnp.int32(0)
    outs = [None] * NCH
    for c in range(NCH):
        if c >= 2:
            outs[c - 2].wait()
        g = pltpu.make_async_copy(
            mem_hbm.at[idx_v.at[pl.ds(base + c * GC, GC)]], bufs[c % 2], gsem)
        g.start()
        cnt = lax.fori_loop(c * SPC, (c + 1) * SPC, scan_body, cnt)
        g.wait()
        o = pltpu.make_async_copy(
            bufs[c % 2], h_hbm.at[pl.ds(base + c * GC, GC)], osem)
        o.start()
        outs[c] = o

    nchunk = (cnt + 15) // 16

    def claim_body(t, _):
        p = t * 16
        pk = plist[pl.ds(p, 16)]
        valid = lane < (cnt - p)
        pk = jnp.where(valid, pk, INVALID)
        sk, _sv = plsc.sort_key_val(pk, pk)
        grp = lax.shift_right_logical(sk, EBITS)
        nxt = grp[jnp.minimum(lane + 1, 15)]
        win = ((grp != nxt) | (lane == 15)) & (sk != INVALID)
        plsc.store_scatter(claim, [grp], sk & (B - 1), mask=win)
        return 0

    lax.fori_loop(0, nchunk, claim_body, 0)

    def winner_body(t, wcnt):
        p = t * 16
        pk = plist[pl.ds(p, 16)]
        valid = lane < (cnt - p)
        rrel = jnp.where(valid, lax.shift_right_logical(pk, EBITS), 0)
        cl = plsc.load_gather(claim, [rrel])
        win = valid & (cl == (pk & (B - 1)))
        wc = plsc.cumsum(jnp.where(win, jnp.int32(1), jnp.int32(0)))
        plsc.store_scatter(wlist, [16 + wcnt + wc - 1], pk, mask=win)
        return wcnt + wc[15]

    wcnt = lax.fori_loop(0, nchunk, winner_body, jnp.int32(0))
    wlist[pl.ds(0, 16)] = jnp.broadcast_to(wcnt, (16,))

    pad = jnp.broadcast_to(wlist[pl.ds(16, 16)][0], (16,))

    def pad_body(t, _):
        wlist[pl.ds(16 + wcnt + t * 16, 16)] = pad
        return 0

    lax.fori_loop(0, SCCH // 16, pad_body, 0)

    nwords = 16 + ((wcnt + SCCH - 1) // SCCH) * SCCH

    def dump_body(t, _):
        pltpu.sync_copy(wlist.at[pl.ds(t * 2048, 2048)],
                        wl_hbm.at[pl.ds(wid * WL + t * 2048, 2048)])
        return 0

    lax.fori_loop(0, (nwords + 2047) // 2048, dump_body, 0)

    for c in range(max(0, NCH - 2), NCH):
        outs[c].wait()


def _tc_body(raw_ref, h_ref, mem_ref, W1_ref, b1_ref, W2_ref, b2_ref,
             Wx_ref, Wh_ref, bx_ref, bh_ref, out_mem_ref, h_new_ref):
    out_mem_ref[...] = mem_ref[...]
    f32 = jnp.float32
    bf = jnp.bfloat16
    x = jnp.maximum(
        lax.dot(raw_ref[...].astype(bf), W1_ref[...].astype(bf),
                preferred_element_type=f32) + b1_ref[...], 0.0)
    msg = lax.dot(x.astype(bf), W2_ref[...].astype(bf),
                  preferred_element_type=f32) + b2_ref[...]
    gx = lax.dot(msg.astype(bf), Wx_ref[...].astype(bf),
                 preferred_element_type=f32) + bx_ref[...]
    h = h_ref[...]
    gh = lax.dot(h.astype(bf), Wh_ref[...].astype(bf),
                 preferred_element_type=f32) + bh_ref[...]
    xr, xz, xn = gx[:, :D], gx[:, D:2 * D], gx[:, 2 * D:]
    hr, hz, hn = gh[:, :D], gh[:, D:2 * D], gh[:, 2 * D:]
    r = jax.nn.sigmoid(xr + hr)
    z = jax.nn.sigmoid(xz + hz)
    n = jnp.tanh(xn + r * hn)
    h_new_ref[...] = (1.0 - z) * n + z * h


def _tc_call(raw_msg, h, mem, W1, b1, W2, b2, Wx, Wh, bx, bh):
    full = lambda s: pl.BlockSpec(s, lambda b: (0, 0))
    return pl.pallas_call(
        _tc_body,
        grid=(GRID,),
        in_specs=[
            pl.BlockSpec((BB, RAW), lambda b: (b, 0)),
            pl.BlockSpec((BB, D), lambda b: (b, 0)),
            pl.BlockSpec((MB, D), lambda b: (b, 0)),
            full((RAW, HID)),
            full((1, HID)),
            full((HID, MSG)),
            full((1, MSG)),
            full((MSG, 3 * D)),
            full((D, 3 * D)),
            full((1, 3 * D)),
            full((1, 3 * D)),
        ],
        out_specs=[
            pl.BlockSpec((MB, D), lambda b: (b, 0)),
            pl.BlockSpec((BB, D), lambda b: (b, 0)),
        ],
        out_shape=[
            jax.ShapeDtypeStruct((N, D), jnp.float32),
            jax.ShapeDtypeStruct((B, D), jnp.float32),
        ],
        compiler_params=pltpu.CompilerParams(
            dimension_semantics=("arbitrary",),
        ),
    )(raw_msg, h, mem, W1, b1, W2, b2, Wx, Wh, bx, bh)


@functools.partial(
    pl.kernel,
    out_type=(),
    mesh=_sc_mesh,
    scratch_types=[
        pltpu.VMEM((16,), jnp.int32),
        pltpu.VMEM((SCCH,), jnp.int32),
        pltpu.VMEM((SCCH,), jnp.int32),
        pltpu.VMEM((SCCH,), jnp.int32),
        pltpu.VMEM((SCCH, D), jnp.float32),
    ],
    compiler_params=pltpu.CompilerParams(needs_layout_passes=False),
)
def _sc_scatter(out_hbm, hnew_hbm, wl_hbm, hdr, wchunk, eidx, ridx, rowbuf):
    wid = lax.axis_index("s") * NC + lax.axis_index("c")
    lo = wid * RPW

    pltpu.sync_copy(wl_hbm.at[pl.ds(wid * WL, 16)], hdr)
    wcnt = hdr[pl.ds(0, 16)][0]

    def chunk_body(t, _):
        pltpu.sync_copy(
            wl_hbm.at[pl.ds(wid * WL + 16 + t * SCCH, SCCH)], wchunk)

        def dec_body(k, _):
            pk = wchunk[pl.ds(k * 16, 16)]
            eidx[pl.ds(k * 16, 16)] = pk & (B - 1)
            ridx[pl.ds(k * 16, 16)] = lax.shift_right_logical(pk, EBITS) + lo
            return 0

        lax.fori_loop(0, SCCH // 16, dec_body, 0, unroll=True)
        pltpu.sync_copy(hnew_hbm.at[eidx], rowbuf)
        pltpu.sync_copy(rowbuf, out_hbm.at[ridx])
        return 0

    lax.fori_loop(0, (wcnt + SCCH - 1) // SCCH, chunk_body, 0)


def kernel(mem, idx, raw_msg, W1, b1, W2, b2, Wx, Wh, bx, bh):
    h, wl = _sc_prep(mem, idx)

    out_mem, h_new = _tc_call(raw_msg, h, mem, W1, b1.reshape(1, -1), W2,
                              b2.reshape(1, -1), Wx, Wh, bx.reshape(1, -1),
                              bh.reshape(1, -1))

    out_ref = jax.new_ref(out_mem)
    _sc_scatter(out_ref, h_new, wl)
    return out_ref[...]
